# Initial kernel scaffold; baseline (speedup 1.0000x reference)
import jax
import jax.numpy as jnp
from jax.experimental import pallas as pl


def kernel(x, H, theta, bias):
    raise NotImplementedError("write your pallas kernel here")


# speedup vs baseline: 23.1511x; 23.1511x over previous
import functools

import jax
import jax.numpy as jnp
from jax import lax
from jax.experimental import pallas as pl
from jax.experimental.pallas import tpu as pltpu
from jax.experimental.pallas import tpu_sc as plsc

B = 2
N = 10000
E = 320000
C = 128

NC = 2
NS = 16
LANES = 16

E_PER_TILE = E // NS
CHUNK = 80
N_FULL = E_PER_TILE // CHUNK

HCHUNK = 4000
NH_CHUNKS = E // HCHUNK

RBLK = 40
NBLK_TOT = N // RBLK
BLK_ROUNDS = (NBLK_TOT + NS - 1) // NS


def _matmul_body(x_ref, th_ref, o_ref):
    o_ref[0] = jnp.dot(x_ref[0], th_ref[...], preferred_element_type=jnp.float32)


def _project(x, theta):
    RB = 1000
    return pl.pallas_call(
        _matmul_body,
        grid=(B, N // RB),
        in_specs=[
            pl.BlockSpec((1, RB, C), lambda b, i: (b, i, 0)),
            pl.BlockSpec((C, C), lambda b, i: (0, 0)),
        ],
        out_specs=pl.BlockSpec((1, RB, C), lambda b, i: (b, i, 0)),
        out_shape=jax.ShapeDtypeStruct((B, N, C), jnp.float32),
    )(x, theta)


def _sc_body(xt_hbm, nidx_hbm, eidx_hbm, bias_hbm, out_hbm, xe_hbm,
             acc_sh, idx_src_v, idx_dst_v, rows_v, hist_v, hidx_v,
             nrm_v, bias_v, sem):
    c = lax.axis_index("c")
    s = lax.axis_index("s")
    ebase = s * E_PER_TILE
    row0 = c * N

    pltpu.sync_copy(bias_hbm, bias_v)

    def zero_acc():
        def zrow(r, _):
            for j in range(C // LANES):
                nrm_v[r, j * LANES:(j + 1) * LANES] = jnp.zeros((LANES,), jnp.float32)
            return 0
        lax.fori_loop(0, RBLK, zrow, 0)
        for i in range(BLK_ROUNDS):
            blk = i * NS + s

            @pl.when(blk < NBLK_TOT)
            def _():
                pltpu.sync_copy(nrm_v, acc_sh.at[pl.ds(blk * RBLK, RBLK)])

    def build_hist(dst_hbm):
        def zh(i, _):
            hist_v[pl.ds(i * LANES, LANES)] = jnp.zeros((LANES,), jnp.float32)
            return 0
        lax.fori_loop(0, N // LANES, zh, 0)
        ones16 = jnp.ones((LANES,), jnp.float32)

        def hchunk(h, _):
            pltpu.sync_copy(dst_hbm.at[pl.ds(c * E + h * HCHUNK, HCHUNK)], hidx_v)

            def add_j(j, _):
                iv = hidx_v[pl.ds(j * LANES, LANES)]
                plsc.addupdate_scatter(hist_v, [iv], ones16)
                return 0
            lax.fori_loop(0, HCHUNK // LANES, add_j, 0)
            return 0
        lax.fori_loop(0, NH_CHUNKS, hchunk, 0)

    def stream_pass(tbl_hbm, src_hbm, dst_hbm):
        def chunk_body(k, _):
            off = c * E + ebase + k * CHUNK
            pltpu.sync_copy(src_hbm.at[pl.ds(off, CHUNK)], idx_src_v)
            pltpu.sync_copy(dst_hbm.at[pl.ds(off, CHUNK)], idx_dst_v)

            def bias_j(j, _):
                sl = pl.ds(j * LANES, LANES)
                idx_src_v[sl] = idx_src_v[sl] + row0
                return 0
            lax.fori_loop(0, CHUNK // LANES, bias_j, 0)

            pltpu.async_copy(tbl_hbm.at[idx_src_v], rows_v, sem).wait()
            pltpu.sync_copy(rows_v, acc_sh.at[idx_dst_v], add=True)
            return 0
        lax.fori_loop(0, N_FULL, chunk_body, 0)

    def normalize(dst_hbm, add_bias):
        for i in range(BLK_ROUNDS):
            blk = i * NS + s

            @pl.when(blk < NBLK_TOT)
            def _():
                base = blk * RBLK
                pltpu.sync_copy(acc_sh.at[pl.ds(base, RBLK)], nrm_v)

                def nrow(r, _):
                    gi = jnp.full((LANES,), base + r, jnp.int32)
                    d = plsc.load_gather(hist_v, [gi])
                    recip = jnp.where(d > 0.0, 1.0 / d, 0.0)
                    for j in range(C // LANES):
                        sl = pl.ds(j * LANES, LANES)
                        v = nrm_v[r, sl] * recip
                        if add_bias:
                            v = v + bias_v[sl]
                        nrm_v[r, sl] = v
                    return 0
                lax.fori_loop(0, RBLK, nrow, 0)
                pltpu.sync_copy(nrm_v, dst_hbm.at[pl.ds(row0 + base, RBLK)])

    zero_acc()
    plsc.subcore_barrier()
    build_hist(eidx_hbm)
    stream_pass(xt_hbm, nidx_hbm, eidx_hbm)
    plsc.subcore_barrier()
    normalize(xe_hbm, add_bias=False)
    zero_acc()
    plsc.subcore_barrier()
    build_hist(nidx_hbm)
    stream_pass(xe_hbm, eidx_hbm, nidx_hbm)
    plsc.subcore_barrier()
    normalize(out_hbm, add_bias=True)


def _build_sc_kernel(interpret=False):
    mesh = plsc.VectorSubcoreMesh(
        core_axis_name="c", subcore_axis_name="s", num_cores=NC, num_subcores=NS
    )
    return pl.kernel(
        _sc_body,
        out_type=(
            jax.ShapeDtypeStruct((B * N, C), jnp.float32),
            jax.ShapeDtypeStruct((B * N, C), jnp.float32),
        ),
        mesh=mesh,
        compiler_params=pltpu.CompilerParams(needs_layout_passes=False),
        scratch_types=[
            pltpu.VMEM_SHARED((N, C), jnp.float32),
            pltpu.VMEM((CHUNK,), jnp.int32),
            pltpu.VMEM((CHUNK,), jnp.int32),
            pltpu.VMEM((CHUNK, C), jnp.float32),
            pltpu.VMEM((N,), jnp.float32),
            pltpu.VMEM((HCHUNK,), jnp.int32),
            pltpu.VMEM((RBLK, C), jnp.float32),
            pltpu.VMEM((C,), jnp.float32),
            pltpu.SemaphoreType.DMA,
        ],
        interpret=interpret,
    )


_hyconv_sc = _build_sc_kernel()


def kernel(x, H, theta, bias):
    xt = _project(x, theta).reshape(B * N, C)
    nidx = H[:, 0, :].reshape(-1)
    eidx = H[:, 1, :].reshape(-1)
    out, _ = _hyconv_sc(xt, nidx, eidx, bias)
    return out.reshape(B, N, C)
